# baseline (speedup 1.0000x reference)
import functools

import jax
import jax.numpy as jnp
from jax import lax
from jax.experimental import pallas as pl
from jax.experimental.pallas import tpu as pltpu
from jax.experimental.pallas import tpu_sc as plsc

VOCAB_SIZE = 1000000
EMB_SIZE = 32
BATCH = 16384
HIST_LEN = 50


def _make_gather(V, D, B, H):
    info = plsc.get_sparse_core_info()
    NC, NS = info.num_cores, info.num_subcores
    NW = NC * NS
    bw = B // NW
    nc_blk = D // 8
    nb_blk = bw // 128
    C0_mul = nb_blk
    assert H % 2 == 0
    mesh = plsc.VectorSubcoreMesh(core_axis_name="c", subcore_axis_name="s")

    @functools.partial(
        pl.kernel,
        mesh=mesh,
        out_type=jax.ShapeDtypeStruct((H, nc_blk, B // 128, 8, 128),
                                      jnp.float32),
        scratch_types=[
            pltpu.VMEM((2, bw), jnp.int32),
            pltpu.VMEM((2, bw, D), jnp.float32),
            pltpu.VMEM((2, nc_blk, nb_blk, 8, 128), jnp.float32),
            pltpu.SemaphoreType.DMA,
            pltpu.SemaphoreType.DMA,
            pltpu.SemaphoreType.DMA,
            pltpu.SemaphoreType.DMA,
            pltpu.SemaphoreType.DMA,
            pltpu.SemaphoreType.DMA,
        ],
        compiler_params=pltpu.CompilerParams(
            use_tc_tiling_on_sc=False, needs_layout_passes=False),
    )
    def k(tok_hbm, tab_hbm, out_hbm, idx_v, rows_v, t_v,
          si0, si1, sg0, sg1, so0, so1):
        wid = lax.axis_index("s") * NC + lax.axis_index("c")
        b0 = wid * bw
        C0 = wid * C0_mul
        si = (si0, si1)
        sg = (sg0, sg1)
        so = (so0, so1)
        iota = lax.iota(jnp.int32, 16)

        def idx_copy(h, b):
            return pltpu.make_async_copy(
                tok_hbm.at[h, pl.ds(b0, bw)], idx_v.at[b], si[b])

        def gather(b):
            return pltpu.make_async_copy(
                tab_hbm.at[idx_v.at[b]], rows_v.at[b], sg[b])

        def store(h, b):
            return pltpu.make_async_copy(
                t_v.at[b], out_hbm.at[h, :, pl.ds(C0, nb_blk)], so[b])

        def transpose(b):
            rows_b = rows_v.at[b]
            for R in range(nc_blk):
                t_R = t_v.at[b, R]

                @plsc.parallel_loop(0, nb_blk * 64, unroll=8)
                def body(kk, _R=R, _t=t_R):
                    cb = kk >> 6
                    r = (kk >> 3) & 7
                    lc = kk & 7
                    tok_idx = cb * 128 + lc * 16 + iota
                    c_idx = jnp.full((16,), _R * 8 + r, jnp.int32)
                    vals = plsc.load_gather(rows_b, [tok_idx, c_idx])
                    _t[cb, r, pl.ds(lc * 16, 16)] = vals

        def step(h, b, first=False, pre_gather=True, pre_idx=True):
            b1 = 1 - b
            if pre_gather:
                idx_copy(h + 1, b1).wait()
                gather(b1).start()
            gather(b).wait()
            if not first:
                store(h - 2, b).wait()
            transpose(b)
            store(h, b).start()
            if pre_idx:
                idx_copy(h + 2, b).start()

        idx_copy(0, 0).start()
        idx_copy(1, 1).start()
        idx_copy(0, 0).wait()
        gather(0).start()
        step(0, 0, first=True)
        step(1, 1, first=True)

        def body(o, carry):
            step(2 * o, 0)
            step(2 * o + 1, 1)
            return carry

        lax.fori_loop(1, H // 2 - 1, body, 0)

        step(H - 2, 0, pre_gather=True, pre_idx=False)
        step(H - 1, 1, pre_gather=False, pre_idx=False)
        store(H - 2, 0).wait()
        store(H - 1, 1).wait()

    return k


_gather = _make_gather(VOCAB_SIZE, EMB_SIZE, BATCH, HIST_LEN)


def _make_transpose(V, D):
    info = plsc.get_sparse_core_info()
    NC, NS = info.num_cores, info.num_subcores
    NW = NC * NS
    KB = 1000
    NBLK = V // KB
    NIT = -(-NBLK // NW)
    assert NIT % 2 == 0
    mesh = plsc.VectorSubcoreMesh(core_axis_name="c", subcore_axis_name="s")

    @functools.partial(
        pl.kernel,
        mesh=mesh,
        out_type=jax.ShapeDtypeStruct((V, D), jnp.float32),
        scratch_types=[
            pltpu.VMEM((2, D, KB), jnp.float32),
            pltpu.VMEM((2, KB, D), jnp.float32),
            pltpu.SemaphoreType.DMA,
            pltpu.SemaphoreType.DMA,
            pltpu.SemaphoreType.DMA,
            pltpu.SemaphoreType.DMA,
        ],
        compiler_params=pltpu.CompilerParams(
            use_tc_tiling_on_sc=False, needs_layout_passes=False),
    )
    def k(tt_hbm, tr_hbm, in_v, out_v, si0, si1, so0, so1):
        wid = lax.axis_index("s") * NC + lax.axis_index("c")
        si = (si0, si1)
        so = (so0, so1)
        iota = lax.iota(jnp.int32, 16)

        def v_base(i):
            blk = wid + i * NW
            blk = jnp.where(blk < NBLK, blk, wid)
            return blk * KB

        def in_copy(i, b):
            return pltpu.make_async_copy(
                tt_hbm.at[:, pl.ds(v_base(i), KB)], in_v.at[b], si[b])

        def out_copy(i, b):
            return pltpu.make_async_copy(
                out_v.at[b], tr_hbm.at[pl.ds(v_base(i), KB)], so[b])

        def transpose(b):
            in_b = in_v.at[b]
            out_b = out_v.at[b]

            @plsc.parallel_loop(0, 2 * KB, unroll=8)
            def body(kk):
                v = kk >> 1
                half = kk & 1
                c_idx = half * 16 + iota
                vals = plsc.load_gather(
                    in_b, [c_idx, jnp.full((16,), v, jnp.int32)])
                out_b[v, pl.ds(half * 16, 16)] = vals

        def step(i, b, first=False, pre=True):
            in_copy(i, b).wait()
            if not first:
                out_copy(i - 2, b).wait()
            transpose(b)
            out_copy(i, b).start()
            if pre:
                in_copy(i + 2, b).start()

        in_copy(0, 0).start()
        in_copy(1, 1).start()
        step(0, 0, first=True)
        step(1, 1, first=True)

        def body(o, carry):
            step(2 * o, 0)
            step(2 * o + 1, 1)
            return carry

        lax.fori_loop(1, NIT // 2 - 1, body, 0)

        step(NIT - 2, 0, pre=False)
        step(NIT - 1, 1, pre=False)
        out_copy(NIT - 2, 0).wait()
        out_copy(NIT - 1, 1).wait()

    return k


_transpose = _make_transpose(VOCAB_SIZE, EMB_SIZE)


def kernel(tokens, embedding_weight):
    tok_t = tokens.T.astype(jnp.int32)
    tt128 = lax.optimization_barrier(
        embedding_weight.T.reshape(VOCAB_SIZE * EMB_SIZE // 128, 128))
    tt = tt128.reshape(EMB_SIZE, VOCAB_SIZE)
    tab = _transpose(tt)
    out5 = _gather(tok_t, tab)
    out = out5.transpose(2, 4, 0, 1, 3).reshape(BATCH, HIST_LEN, EMB_SIZE)
    return out


# speedup vs baseline: 1.3131x; 1.3131x over previous
import functools

import jax
import jax.numpy as jnp
from jax import lax
from jax.experimental import pallas as pl
from jax.experimental.pallas import tpu as pltpu
from jax.experimental.pallas import tpu_sc as plsc

VOCAB_SIZE = 1000000
EMB_SIZE = 32
BATCH = 16384
HIST_LEN = 50


def _make_gather(V, D, B, H):
    info = plsc.get_sparse_core_info()
    NC, NS = info.num_cores, info.num_subcores
    NW = NC * NS
    bw = B // NW
    nc_blk = D // 8
    nb_blk = bw // 128
    C0_mul = nb_blk
    assert H % 2 == 0
    mesh = plsc.VectorSubcoreMesh(core_axis_name="c", subcore_axis_name="s")

    @functools.partial(
        pl.kernel,
        mesh=mesh,
        out_type=jax.ShapeDtypeStruct((H, nc_blk, B // 128, 8, 128),
                                      jnp.float32),
        scratch_types=[
            pltpu.VMEM((2, bw), jnp.int32),
            pltpu.VMEM((2, bw, D), jnp.float32),
            pltpu.VMEM((2, nc_blk, nb_blk, 8, 128), jnp.float32),
            pltpu.SemaphoreType.DMA,
            pltpu.SemaphoreType.DMA,
            pltpu.SemaphoreType.DMA,
            pltpu.SemaphoreType.DMA,
            pltpu.SemaphoreType.DMA,
            pltpu.SemaphoreType.DMA,
        ],
        compiler_params=pltpu.CompilerParams(
            use_tc_tiling_on_sc=False, needs_layout_passes=False),
    )
    def k(tok_hbm, tab_hbm, out_hbm, idx_v, rows_v, t_v,
          si0, si1, sg0, sg1, so0, so1):
        wid = lax.axis_index("s") * NC + lax.axis_index("c")
        b0 = wid * bw
        C0 = wid * C0_mul
        si = (si0, si1)
        sg = (sg0, sg1)
        so = (so0, so1)
        iota = lax.iota(jnp.int32, 16)

        def idx_copy(h, b):
            return pltpu.make_async_copy(
                tok_hbm.at[h, pl.ds(b0, bw)], idx_v.at[b], si[b])

        def gather(b):
            return pltpu.make_async_copy(
                tab_hbm.at[idx_v.at[b]], rows_v.at[b], sg[b])

        def store(h, b):
            return pltpu.make_async_copy(
                t_v.at[b], out_hbm.at[h, :, pl.ds(C0, nb_blk)], so[b])

        def transpose(b):
            rows_b = rows_v.at[b]
            for R in range(nc_blk):
                t_R = t_v.at[b, R]

                @plsc.parallel_loop(0, nb_blk * 64, unroll=8)
                def body(kk, _R=R, _t=t_R):
                    cb = kk >> 6
                    r = (kk >> 3) & 7
                    lc = kk & 7
                    tok_idx = cb * 128 + lc * 16 + iota
                    c_idx = jnp.full((16,), _R * 8 + r, jnp.int32)
                    vals = plsc.load_gather(rows_b, [tok_idx, c_idx])
                    _t[cb, r, pl.ds(lc * 16, 16)] = vals

        def step(h, b, first=False, pre_gather=True, pre_idx=True):
            b1 = 1 - b
            if pre_gather:
                idx_copy(h + 1, b1).wait()
                gather(b1).start()
            gather(b).wait()
            if not first:
                store(h - 2, b).wait()
            transpose(b)
            store(h, b).start()
            if pre_idx:
                idx_copy(h + 2, b).start()

        idx_copy(0, 0).start()
        idx_copy(1, 1).start()
        idx_copy(0, 0).wait()
        gather(0).start()
        step(0, 0, first=True)
        step(1, 1, first=True)

        def body(o, carry):
            step(2 * o, 0)
            step(2 * o + 1, 1)
            return carry

        lax.fori_loop(1, H // 2 - 1, body, 0)

        step(H - 2, 0, pre_gather=True, pre_idx=False)
        step(H - 1, 1, pre_gather=False, pre_idx=False)
        store(H - 2, 0).wait()
        store(H - 1, 1).wait()

    return k


_gather = _make_gather(VOCAB_SIZE, EMB_SIZE, BATCH, HIST_LEN)


def _make_transpose(V, D):
    info = plsc.get_sparse_core_info()
    NC, NS = info.num_cores, info.num_subcores
    NW = NC * NS
    KB = 1000
    NBLK = V // KB
    NIT = -(-NBLK // NW)
    assert NIT % 2 == 0
    mesh = plsc.VectorSubcoreMesh(core_axis_name="c", subcore_axis_name="s")

    @functools.partial(
        pl.kernel,
        mesh=mesh,
        out_type=jax.ShapeDtypeStruct((V, D), jnp.float32),
        scratch_types=[
            pltpu.VMEM((2, D, KB), jnp.float32),
            pltpu.VMEM((2, KB, D), jnp.float32),
            pltpu.SemaphoreType.DMA,
            pltpu.SemaphoreType.DMA,
            pltpu.SemaphoreType.DMA,
            pltpu.SemaphoreType.DMA,
        ],
        compiler_params=pltpu.CompilerParams(
            use_tc_tiling_on_sc=False, needs_layout_passes=False),
    )
    def k(tt_hbm, tr_hbm, in_v, out_v, si0, si1, so0, so1):
        wid = lax.axis_index("s") * NC + lax.axis_index("c")
        si = (si0, si1)
        so = (so0, so1)
        iota = lax.iota(jnp.int32, 16)

        def v_base(i):
            blk = wid + i * NW
            blk = jnp.where(blk < NBLK, blk, wid)
            return blk * KB

        def in_copy(i, b):
            return pltpu.make_async_copy(
                tt_hbm.at[:, pl.ds(v_base(i), KB)], in_v.at[b], si[b])

        def out_copy(i, b):
            return pltpu.make_async_copy(
                out_v.at[b], tr_hbm.at[pl.ds(v_base(i), KB)], so[b])

        def transpose(b):
            in_b = in_v.at[b]
            out_b = out_v.at[b]

            @plsc.parallel_loop(0, 2 * KB, unroll=8)
            def body(kk):
                v = kk >> 1
                half = kk & 1
                c_idx = half * 16 + iota
                vals = plsc.load_gather(
                    in_b, [c_idx, jnp.full((16,), v, jnp.int32)])
                out_b[v, pl.ds(half * 16, 16)] = vals

        def step(i, b, first=False, pre=True):
            in_copy(i, b).wait()
            if not first:
                out_copy(i - 2, b).wait()
            transpose(b)
            out_copy(i, b).start()
            if pre:
                in_copy(i + 2, b).start()

        in_copy(0, 0).start()
        in_copy(1, 1).start()
        step(0, 0, first=True)
        step(1, 1, first=True)

        def body(o, carry):
            step(2 * o, 0)
            step(2 * o + 1, 1)
            return carry

        lax.fori_loop(1, NIT // 2 - 1, body, 0)

        step(NIT - 2, 0, pre=False)
        step(NIT - 1, 1, pre=False)
        out_copy(NIT - 2, 0).wait()
        out_copy(NIT - 1, 1).wait()

    return k


_transpose = _make_transpose(VOCAB_SIZE, EMB_SIZE)


def kernel(tokens, embedding_weight):
    tok_t = tokens.T.astype(jnp.int32)
    tab = _transpose(embedding_weight.T)
    out5 = _gather(tok_t, tab)
    out = out5.transpose(2, 4, 0, 1, 3).reshape(BATCH, HIST_LEN, EMB_SIZE)
    return out


# speedup vs baseline: 4.0814x; 3.1082x over previous
import functools

import jax
import jax.numpy as jnp
from jax import lax
from jax.experimental import pallas as pl
from jax.experimental.pallas import tpu as pltpu
from jax.experimental.pallas import tpu_sc as plsc

VOCAB_SIZE = 1000000
EMB_SIZE = 32
BATCH = 16384
HIST_LEN = 50


def _make_gather(V, D, B, H, PD=128, CH=256):
    info = plsc.get_sparse_core_info()
    NC, NS = info.num_cores, info.num_subcores
    NW = NC * NS
    bw = B // NW
    nh = bw // CH
    NU = H * nh
    nc_blk = D // 8
    nb_blk = CH // 128
    assert NU % 2 == 0
    mesh = plsc.VectorSubcoreMesh(core_axis_name="c", subcore_axis_name="s")

    @functools.partial(
        pl.kernel,
        mesh=mesh,
        out_type=jax.ShapeDtypeStruct((H, nc_blk, B // 128, 8, 128),
                                      jnp.float32),
        scratch_types=[
            pltpu.VMEM((2, CH), jnp.int32),
            pltpu.VMEM((2, CH, PD), jnp.float32),
            pltpu.VMEM((2, nc_blk, nb_blk, 8, 128), jnp.float32),
            pltpu.SemaphoreType.DMA,
            pltpu.SemaphoreType.DMA,
            pltpu.SemaphoreType.DMA,
            pltpu.SemaphoreType.DMA,
            pltpu.SemaphoreType.DMA,
            pltpu.SemaphoreType.DMA,
        ],
        compiler_params=pltpu.CompilerParams(
            use_tc_tiling_on_sc=False, needs_layout_passes=False),
    )
    def k(tok_hbm, tab_hbm, out_hbm, idx_v, rows_v, t_v,
          si0, si1, sg0, sg1, so0, so1):
        wid = lax.axis_index("s") * NC + lax.axis_index("c")
        si = (si0, si1)
        sg = (sg0, sg1)
        so = (so0, so1)
        iota = lax.iota(jnp.int32, 16)

        def hb(u):
            h = u // nh
            hh = u % nh
            return h, wid * bw + hh * CH, wid * (bw // 128) + hh * nb_blk

        def idx_copy(u, b):
            h, t0, _ = hb(u)
            return pltpu.make_async_copy(
                tok_hbm.at[h, pl.ds(t0, CH)], idx_v.at[b], si[b])

        def gather(b):
            return pltpu.make_async_copy(
                tab_hbm.at[idx_v.at[b]], rows_v.at[b], sg[b])

        def store(u, b):
            h, _, c0 = hb(u)
            return pltpu.make_async_copy(
                t_v.at[b], out_hbm.at[h, :, pl.ds(c0, nb_blk)], so[b])

        def transpose(b):
            rows_b = rows_v.at[b]
            for R in range(nc_blk):
                t_R = t_v.at[b, R]

                @plsc.parallel_loop(0, nb_blk * 64, unroll=8)
                def body(kk, _R=R, _t=t_R):
                    cb = kk >> 6
                    r = (kk >> 3) & 7
                    lc = kk & 7
                    tok_idx = cb * 128 + lc * 16 + iota
                    c_idx = jnp.full((16,), _R * 8 + r, jnp.int32)
                    vals = plsc.load_gather(rows_b, [tok_idx, c_idx])
                    _t[cb, r, pl.ds(lc * 16, 16)] = vals

        def step(u, b, first=False, pre_gather=True, pre_idx=True):
            b1 = 1 - b
            if pre_gather:
                idx_copy(u + 1, b1).wait()
                gather(b1).start()
            gather(b).wait()
            if not first:
                store(u - 2, b).wait()
            transpose(b)
            store(u, b).start()
            if pre_idx:
                idx_copy(u + 2, b).start()

        idx_copy(0, 0).start()
        idx_copy(1, 1).start()
        idx_copy(0, 0).wait()
        gather(0).start()
        step(0, 0, first=True)
        step(1, 1, first=True)

        def body(o, carry):
            step(2 * o, 0)
            step(2 * o + 1, 1)
            return carry

        lax.fori_loop(1, NU // 2 - 1, body, 0)

        step(NU - 2, 0, pre_gather=True, pre_idx=False)
        step(NU - 1, 1, pre_gather=False, pre_idx=False)
        store(NU - 2, 0).wait()
        store(NU - 1, 1).wait()

    return k


_gather = _make_gather(VOCAB_SIZE, EMB_SIZE, BATCH, HIST_LEN)


def kernel(tokens, embedding_weight):
    tok_t = tokens.T.astype(jnp.int32)
    tab = jnp.pad(embedding_weight, ((0, 0), (0, 128 - EMB_SIZE)))
    out5 = _gather(tok_t, tab)
    out = out5.transpose(2, 4, 0, 1, 3).reshape(BATCH, HIST_LEN, EMB_SIZE)
    return out


# speedup vs baseline: 4.1979x; 1.0285x over previous
import functools

import jax
import jax.numpy as jnp
from jax import lax
from jax.experimental import pallas as pl
from jax.experimental.pallas import tpu as pltpu
from jax.experimental.pallas import tpu_sc as plsc

VOCAB_SIZE = 1000000
EMB_SIZE = 32
BATCH = 16384
HIST_LEN = 50


def _make_gather(V, D, B, H, PD=128, CH=256):
    info = plsc.get_sparse_core_info()
    NC, NS = info.num_cores, info.num_subcores
    NW = NC * NS
    bw = B // NW
    nh = bw // CH
    NU = H * nh
    nc_blk = D // 8
    nb_blk = CH // 128
    assert NU % 2 == 0
    mesh = plsc.VectorSubcoreMesh(core_axis_name="c", subcore_axis_name="s")

    @functools.partial(
        pl.kernel,
        mesh=mesh,
        out_type=jax.ShapeDtypeStruct((H, nc_blk, B // 128, 8, 128),
                                      jnp.float32),
        scratch_types=[
            pltpu.VMEM((2, CH), jnp.int32),
            pltpu.VMEM((2, CH, PD), jnp.float32),
            pltpu.VMEM((2, nc_blk, nb_blk, 8, 128), jnp.float32),
            pltpu.SemaphoreType.DMA,
            pltpu.SemaphoreType.DMA,
            pltpu.SemaphoreType.DMA,
            pltpu.SemaphoreType.DMA,
            pltpu.SemaphoreType.DMA,
            pltpu.SemaphoreType.DMA,
        ],
        compiler_params=pltpu.CompilerParams(
            use_tc_tiling_on_sc=False, needs_layout_passes=False),
    )
    def k(tok_hbm, tab_hbm, out_hbm, idx_v, rows_v, t_v,
          si0, si1, sg0, sg1, so0, so1):
        wid = lax.axis_index("s") * NC + lax.axis_index("c")
        si = (si0, si1)
        sg = (sg0, sg1)
        so = (so0, so1)
        iota = lax.iota(jnp.int32, 16)

        def hb(u):
            h = u // nh
            hh = u % nh
            return h, wid * bw + hh * CH, wid * (bw // 128) + hh * nb_blk

        def idx_copy(u, b):
            h, t0, _ = hb(u)
            return pltpu.make_async_copy(
                tok_hbm.at[h, pl.ds(t0, CH)], idx_v.at[b], si[b])

        def gather(b):
            return pltpu.make_async_copy(
                tab_hbm.at[idx_v.at[b]], rows_v.at[b], sg[b])

        def store(u, b):
            h, _, c0 = hb(u)
            return pltpu.make_async_copy(
                t_v.at[b], out_hbm.at[h, :, pl.ds(c0, nb_blk)], so[b])

        def transpose(b):
            rows_b = rows_v.at[b]
            for R in range(nc_blk):
                t_R = t_v.at[b, R]

                @plsc.parallel_loop(0, nb_blk * 64, unroll=16)
                def body(kk, _R=R, _t=t_R):
                    cb = kk >> 6
                    r = (kk >> 3) & 7
                    lc = kk & 7
                    tok_idx = cb * 128 + lc * 16 + iota
                    c_idx = jnp.full((16,), _R * 8 + r, jnp.int32)
                    vals = plsc.load_gather(rows_b, [tok_idx, c_idx])
                    _t[cb, r, pl.ds(lc * 16, 16)] = vals

        def step(u, b, first=False, pre_gather=True, pre_idx=True):
            b1 = 1 - b
            if pre_gather:
                idx_copy(u + 1, b1).wait()
                gather(b1).start()
            gather(b).wait()
            if not first:
                store(u - 2, b).wait()
            transpose(b)
            store(u, b).start()
            if pre_idx:
                idx_copy(u + 2, b).start()

        idx_copy(0, 0).start()
        idx_copy(1, 1).start()
        idx_copy(0, 0).wait()
        gather(0).start()
        step(0, 0, first=True)
        step(1, 1, first=True)

        def body(o, carry):
            step(2 * o, 0)
            step(2 * o + 1, 1)
            return carry

        lax.fori_loop(1, NU // 2 - 1, body, 0)

        step(NU - 2, 0, pre_gather=True, pre_idx=False)
        step(NU - 1, 1, pre_gather=False, pre_idx=False)
        store(NU - 2, 0).wait()
        store(NU - 1, 1).wait()

    return k


_gather = _make_gather(VOCAB_SIZE, EMB_SIZE, BATCH, HIST_LEN)


def kernel(tokens, embedding_weight):
    tok_t = tokens.T.astype(jnp.int32)
    tab = jnp.pad(embedding_weight, ((0, 0), (0, 128 - EMB_SIZE)))
    out5 = _gather(tok_t, tab)
    out = out5.transpose(2, 4, 0, 1, 3).reshape(BATCH, HIST_LEN, EMB_SIZE)
    return out
